# baseline (speedup 1.0000x reference)
import functools

import jax
import jax.numpy as jnp
from jax import lax
from jax.experimental import pallas as pl
from jax.experimental.pallas import tpu as pltpu
from jax.experimental.pallas import tpu_sc as plsc

G = 2
NV = 10001
E = 160000
H = 128
NPAD = 10240
ROWS = 16
FLAT = ROWS * NPAD
NC = 2
NS = 16
EPAD = 163840
EPT = EPAD // NS
CH = EPT
NPT = NPAD // NS
FPT = FLAT // NS


def _sc_body(ty_hbm, edges_hbm, out_hbm,
             ty_v, src_v, dst_v, ones_v, zeros_v, idx_a, idx_b, idx_oh, shared,
             sem_ty, sem_e, sem_s):
    g = lax.axis_index("c")
    w = lax.axis_index("s")

    ty_dma = pltpu.async_copy(ty_hbm.at[g], ty_v, sem_ty)
    base = w * EPT
    src_dma = pltpu.async_copy(edges_hbm.at[g, 0, pl.ds(base, EPT)], src_v,
                               sem_e)
    dst_dma = pltpu.async_copy(edges_hbm.at[g, 1, pl.ds(base, EPT)], dst_v,
                               sem_e)

    def fill_ones(i, _):
        ones_v[pl.ds(i * 16, 16)] = jnp.ones((16,), jnp.float32)
        return 0
    lax.fori_loop(0, CH // 16, fill_ones, 0)

    def fill_zeros(i, _):
        zeros_v[pl.ds(i * 16, 16)] = jnp.zeros((16,), jnp.float32)
        return 0
    lax.fori_loop(0, FPT // 16, fill_zeros, 0)

    pltpu.sync_copy(zeros_v, shared.at[pl.ds(w * FPT, FPT)])
    ty_dma.wait()
    plsc.subcore_barrier()

    src_dma.wait()
    dst_dma.wait()
    scats = []
    for h, iv in enumerate((idx_a, idx_b)):
        off = h * (CH // 2)

        def vec(k, _, off=off, iv=iv):
            s16 = src_v[pl.ds(off + k * 16, 16)]
            d16 = dst_v[pl.ds(off + k * 16, 16)]
            t16 = plsc.load_gather(ty_v, [s16])
            iv[pl.ds(k * 16, 16)] = t16 * NPAD + d16
            return 0
        lax.fori_loop(0, CH // 32, vec, 0)
        scats.append(pltpu.async_copy(ones_v.at[pl.ds(0, CH // 2)],
                                      shared.at[iv], sem_s, add=True))

    def oh_vec(k, _):
        v0 = w * NPT + k * 16
        t16 = ty_v[pl.ds(v0, 16)]
        idx_oh[pl.ds(k * 16, 16)] = (t16 + 8) * NPAD + (v0 + lax.iota(jnp.int32, 16))
        return 0
    lax.fori_loop(0, NPT // 16, oh_vec, 0)
    scats.append(pltpu.async_copy(ones_v.at[pl.ds(0, NPT)], shared.at[idx_oh],
                                  sem_s, add=True))

    for sc_ in scats:
        sc_.wait()
    plsc.subcore_barrier()

    pltpu.sync_copy(shared.at[pl.ds(w * FPT, FPT)],
                    out_hbm.at[g, pl.ds(w * FPT, FPT)])


@functools.cache
def _sc_hist():
    return functools.partial(
        pl.kernel,
        out_type=jax.ShapeDtypeStruct((G, FLAT), jnp.float32),
        mesh=plsc.VectorSubcoreMesh(core_axis_name="c", subcore_axis_name="s",
                                    num_cores=NC, num_subcores=NS),
        compiler_params=pltpu.CompilerParams(needs_layout_passes=False),
        scratch_types=[
            pltpu.VMEM((NPAD,), jnp.int32),
            pltpu.VMEM((EPT,), jnp.int32),
            pltpu.VMEM((EPT,), jnp.int32),
            pltpu.VMEM((CH,), jnp.float32),
            pltpu.VMEM((FPT,), jnp.float32),
            pltpu.VMEM((EPT // 2,), jnp.int32),
            pltpu.VMEM((EPT // 2,), jnp.int32),
            pltpu.VMEM((NPT,), jnp.int32),
            pltpu.VMEM_SHARED((FLAT,), jnp.float32),
            pltpu.SemaphoreType.DMA,
            pltpu.SemaphoreType.DMA,
            pltpu.SemaphoreType.DMA,
        ],
    )(_sc_body)


NB = 1024
NBLK = NPAD // NB
NV_LAST = NV - (NBLK - 1) * NB


def _tc_body(scat_ref, t8_ref, wl_ref, wr_ref, bl_ref, gnf_hbm, gef_hbm,
             buf, buf2, sem):
    b = pl.program_id(0)
    slot = lax.rem(b, 2)
    t8 = t8_ref[...]
    dn = (((1,), (1,)), ((), ()))
    twl = lax.dot_general(t8, wl_ref[...], dn, preferred_element_type=jnp.float32)
    twr = lax.dot_general(t8, wr_ref[...], dn, preferred_element_type=jnp.float32)
    rhs = jnp.concatenate(
        [jnp.concatenate([twl, twr], axis=0),
         jnp.concatenate([jnp.zeros((8, H), jnp.float32), t8], axis=0)], axis=1)
    rowid = lax.broadcasted_iota(jnp.int32, (ROWS, 1), 0)
    d0 = (((0,), (0,)), ((), ()))

    def dmas(slot_, blk, rows):
        out = []
        for g in range(G):
            src = buf.at[slot_, g, pl.ds(0, rows), :]
            out.append(pltpu.make_async_copy(
                src.at[:, pl.ds(0, H)],
                gef_hbm.at[pl.ds(blk * NB, rows), g, :], sem.at[slot_]))
            out.append(pltpu.make_async_copy(
                src.at[:, pl.ds(H, H)],
                gnf_hbm.at[pl.ds(blk * NB, rows), g, :], sem.at[slot_]))
        return out

    @pl.when(b >= 2)
    def _():
        for c in dmas(slot, b - 2, NB):
            c.wait()

    bias = jnp.concatenate([bl_ref[...], jnp.zeros((1, H), jnp.float32)], axis=1)
    for g in range(G):
        scat = scat_ref[g]
        cnt = jnp.sum(scat[0:8, :], axis=0, keepdims=True)
        recip = 1.0 / jnp.maximum(cnt, 1.0)
        scaled = scat * jnp.where(rowid < 8, recip, 1.0)
        res = lax.dot_general(
            scaled, rhs, d0, preferred_element_type=jnp.float32) + bias
        buf[slot, g] = res
        buf2[g] = res[NV_LAST - 1:NV_LAST, :]

    @pl.when(b < NBLK - 1)
    def _():
        for c in dmas(slot, b, NB):
            c.start()

    @pl.when(b == NBLK - 1)
    def _():
        vlast = b * NB + NV_LAST - 1
        tail = dmas(slot, b, NV_LAST - 1)
        for g in range(G):
            tail.append(pltpu.make_async_copy(
                buf2.at[g, :, pl.ds(0, H)],
                gef_hbm.at[pl.ds(vlast, 1), g, :], sem.at[slot]))
            tail.append(pltpu.make_async_copy(
                buf2.at[g, :, pl.ds(H, H)],
                gnf_hbm.at[pl.ds(vlast, 1), g, :], sem.at[slot]))
        for c in tail:
            c.start()
        for c in dmas(1 - slot, b - 1, NB):
            c.wait()
        for c in tail:
            c.wait()


def _tc_dense(scat, t8, wl, wr, bl):
    return pl.pallas_call(
        _tc_body,
        grid=(NBLK,),
        in_specs=[
            pl.BlockSpec((G, ROWS, NB), lambda b: (0, 0, b)),
            pl.BlockSpec((8, H), lambda b: (0, 0)),
            pl.BlockSpec((H, H), lambda b: (0, 0)),
            pl.BlockSpec((H, H), lambda b: (0, 0)),
            pl.BlockSpec((1, H), lambda b: (0, 0)),
        ],
        out_specs=[
            pl.BlockSpec(memory_space=pl.ANY),
            pl.BlockSpec(memory_space=pl.ANY),
        ],
        out_shape=[
            jax.ShapeDtypeStruct((NV, G, H), jnp.float32),
            jax.ShapeDtypeStruct((NV, G, H), jnp.float32),
        ],
        scratch_shapes=[
            pltpu.VMEM((2, G, NB, 2 * H), jnp.float32),
            pltpu.VMEM((G, 1, 2 * H), jnp.float32),
            pltpu.SemaphoreType.DMA((2,)),
        ],
    )(scat, t8, wl, wr, bl)


@jax.jit
def kernel(input_nodes, input_edges, node_emb, graph_token_emb, W_l, b_l, W_r):
    input_nodes = input_nodes.astype(jnp.int32)
    edges = input_edges.astype(jnp.int32)

    ty = jnp.concatenate(
        [jnp.full((G, 1), 4, jnp.int32),
         input_nodes,
         jnp.zeros((G, NPAD - NV), jnp.int32)], axis=1)

    r = jnp.arange(EPAD - E, dtype=jnp.int32)
    pad = jnp.stack([r & 8191, NV + (r & 127)], axis=0)
    edges_p = jnp.concatenate(
        [edges, jnp.broadcast_to(pad[None], (G, 2, EPAD - E))], axis=2)

    scat = _sc_hist()(ty, edges_p).reshape(G, ROWS, NPAD)

    t8 = jnp.concatenate(
        [node_emb, graph_token_emb, jnp.zeros((3, H), jnp.float32)], axis=0)
    gnf, gef = _tc_dense(scat, t8, W_l, W_r, b_l.reshape(1, H))
    return jnp.swapaxes(gnf, 0, 1), jnp.swapaxes(gef, 0, 1)


# speedup vs baseline: 1.0106x; 1.0106x over previous
import functools

import jax
import jax.numpy as jnp
from jax import lax
from jax.experimental import pallas as pl
from jax.experimental.pallas import tpu as pltpu
from jax.experimental.pallas import tpu_sc as plsc

G = 2
NV = 10001
N_IN = 10000
E = 160000
H = 128
NPAD = 10240
ROWS = 16
FLAT = ROWS * NPAD
NC = 2
NS = 16
EPT = 10240
CH = 2560
NCH = EPT // CH
TRASH = 15 * NPAD + NV
NPT = NPAD // NS
FPT = FLAT // NS


def _sc_body(nodes_hbm, edges_hbm, out_hbm,
             ty2_v, eb0, eb1, ones_v, zeros_v, ix0, ix1, ix2, ix3, idx_oh,
             shared, sem_ty, sem_e, sem_s):
    eb = (eb0, eb1)
    ix = (ix0, ix1, ix2, ix3)
    g = lax.axis_index("c")
    w = lax.axis_index("s")
    g16 = jnp.zeros((16,), jnp.int32) + g

    ty_dma = pltpu.async_copy(nodes_hbm, ty2_v, sem_ty)
    e_dmas = {}

    def fetch(c):
        ebase = w * EPT + c * CH
        base = jnp.minimum(ebase, E - CH)
        e_dmas[c] = (base, ebase, pltpu.async_copy(
            edges_hbm.at[g, :, pl.ds(base, CH)], eb[c % 2], sem_e))

    fetch(0)

    def fill_ones(i, _):
        ones_v[pl.ds(i * 16, 16)] = jnp.ones((16,), jnp.float32)
        return 0
    lax.fori_loop(0, CH // 16, fill_ones, 0)

    def fill_zeros(i, _):
        zeros_v[pl.ds(i * 16, 16)] = jnp.zeros((16,), jnp.float32)
        return 0
    lax.fori_loop(0, FPT // 16, fill_zeros, 0)

    pltpu.sync_copy(zeros_v, shared.at[pl.ds(w * FPT, FPT)])
    ty_dma.wait()
    plsc.subcore_barrier()

    def node_type(v16):
        idx = jnp.clip(v16 - 1, 0, N_IN - 1)
        t16 = plsc.load_gather(ty2_v, [g16, idx])
        return jnp.where(v16 == 0, 4, t16)

    scats = []
    for c in range(NCH):
        b = c % 2
        if c + 1 < NCH:
            fetch(c + 1)
        base, ebase, dma = e_dmas.pop(c)
        dma.wait()
        sv, dv, iv = eb[b], eb[b], ix[c]

        def vec(k, _, sv=sv, iv=iv, base=base, ebase=ebase):
            lidx = k * 16 + lax.iota(jnp.int32, 16)
            s16 = sv[0, pl.ds(k * 16, 16)]
            d16 = sv[1, pl.ds(k * 16, 16)]
            t16 = node_type(s16)
            gid = base + lidx
            valid = (gid >= ebase) & (gid < E)
            iv[pl.ds(k * 16, 16)] = jnp.where(
                valid, t16 * NPAD + d16, TRASH + (lidx & 127))
            return 0
        lax.fori_loop(0, CH // 16, vec, 0)
        scats.append(pltpu.async_copy(ones_v, shared.at[iv], sem_s, add=True))

    def oh_vec(k, _):
        v16 = w * NPT + k * 16 + lax.iota(jnp.int32, 16)
        idx_oh[pl.ds(k * 16, 16)] = (node_type(v16) + 8) * NPAD + v16
        return 0
    lax.fori_loop(0, NPT // 16, oh_vec, 0)
    scats.append(pltpu.async_copy(ones_v.at[pl.ds(0, NPT)], shared.at[idx_oh],
                                  sem_s, add=True))

    for sc_ in scats:
        sc_.wait()
    plsc.subcore_barrier()

    pltpu.sync_copy(shared.at[pl.ds(w * FPT, FPT)],
                    out_hbm.at[g, pl.ds(w * FPT, FPT)])


@functools.cache
def _sc_hist():
    return functools.partial(
        pl.kernel,
        out_type=jax.ShapeDtypeStruct((G, FLAT), jnp.float32),
        mesh=plsc.VectorSubcoreMesh(core_axis_name="c", subcore_axis_name="s",
                                    num_cores=NC, num_subcores=NS),
        compiler_params=pltpu.CompilerParams(needs_layout_passes=False),
        scratch_types=[
            pltpu.VMEM((G, N_IN), jnp.int32),
            pltpu.VMEM((2, CH), jnp.int32),
            pltpu.VMEM((2, CH), jnp.int32),
            pltpu.VMEM((CH,), jnp.float32),
            pltpu.VMEM((FPT,), jnp.float32),
            pltpu.VMEM((CH,), jnp.int32),
            pltpu.VMEM((CH,), jnp.int32),
            pltpu.VMEM((CH,), jnp.int32),
            pltpu.VMEM((CH,), jnp.int32),
            pltpu.VMEM((NPT,), jnp.int32),
            pltpu.VMEM_SHARED((FLAT,), jnp.float32),
            pltpu.SemaphoreType.DMA,
            pltpu.SemaphoreType.DMA,
            pltpu.SemaphoreType.DMA,
        ],
    )(_sc_body)


NB = 1024
NBLK = NPAD // NB
NV_LAST = NV - (NBLK - 1) * NB


def _tc_body(scat_ref, t8_ref, wl_ref, wr_ref, bl_ref, gnf_hbm, gef_hbm,
             buf, buf2, sem):
    b = pl.program_id(0)
    slot = lax.rem(b, 2)
    t8 = t8_ref[...]
    dn = (((1,), (1,)), ((), ()))
    twl = lax.dot_general(t8, wl_ref[...], dn, preferred_element_type=jnp.float32)
    twr = lax.dot_general(t8, wr_ref[...], dn, preferred_element_type=jnp.float32)
    rhs = jnp.concatenate(
        [jnp.concatenate([twl, twr], axis=0),
         jnp.concatenate([jnp.zeros((8, H), jnp.float32), t8], axis=0)], axis=1)
    rowid = lax.broadcasted_iota(jnp.int32, (ROWS, 1), 0)
    d0 = (((0,), (0,)), ((), ()))

    def dmas(slot_, blk, rows):
        out = []
        for g in range(G):
            src = buf.at[slot_, g, pl.ds(0, rows), :]
            out.append(pltpu.make_async_copy(
                src.at[:, pl.ds(0, H)],
                gef_hbm.at[pl.ds(blk * NB, rows), g, :], sem.at[slot_]))
            out.append(pltpu.make_async_copy(
                src.at[:, pl.ds(H, H)],
                gnf_hbm.at[pl.ds(blk * NB, rows), g, :], sem.at[slot_]))
        return out

    @pl.when(b >= 2)
    def _():
        for c in dmas(slot, b - 2, NB):
            c.wait()

    bias = jnp.concatenate([bl_ref[...], jnp.zeros((1, H), jnp.float32)], axis=1)
    for g in range(G):
        scat = scat_ref[g]
        cnt = jnp.sum(scat[0:8, :], axis=0, keepdims=True)
        recip = 1.0 / jnp.maximum(cnt, 1.0)
        scaled = scat * jnp.where(rowid < 8, recip, 1.0)
        res = lax.dot_general(
            scaled, rhs, d0, preferred_element_type=jnp.float32) + bias
        buf[slot, g] = res
        buf2[g] = res[NV_LAST - 1:NV_LAST, :]

    @pl.when(b < NBLK - 1)
    def _():
        for c in dmas(slot, b, NB):
            c.start()

    @pl.when(b == NBLK - 1)
    def _():
        vlast = b * NB + NV_LAST - 1
        tail = dmas(slot, b, NV_LAST - 1)
        for g in range(G):
            tail.append(pltpu.make_async_copy(
                buf2.at[g, :, pl.ds(0, H)],
                gef_hbm.at[pl.ds(vlast, 1), g, :], sem.at[slot]))
            tail.append(pltpu.make_async_copy(
                buf2.at[g, :, pl.ds(H, H)],
                gnf_hbm.at[pl.ds(vlast, 1), g, :], sem.at[slot]))
        for c in tail:
            c.start()
        for c in dmas(1 - slot, b - 1, NB):
            c.wait()
        for c in tail:
            c.wait()


def _tc_dense(scat, t8, wl, wr, bl):
    return pl.pallas_call(
        _tc_body,
        grid=(NBLK,),
        in_specs=[
            pl.BlockSpec((G, ROWS, NB), lambda b: (0, 0, b)),
            pl.BlockSpec((8, H), lambda b: (0, 0)),
            pl.BlockSpec((H, H), lambda b: (0, 0)),
            pl.BlockSpec((H, H), lambda b: (0, 0)),
            pl.BlockSpec((1, H), lambda b: (0, 0)),
        ],
        out_specs=[
            pl.BlockSpec(memory_space=pl.ANY),
            pl.BlockSpec(memory_space=pl.ANY),
        ],
        out_shape=[
            jax.ShapeDtypeStruct((NV, G, H), jnp.float32),
            jax.ShapeDtypeStruct((NV, G, H), jnp.float32),
        ],
        scratch_shapes=[
            pltpu.VMEM((2, G, NB, 2 * H), jnp.float32),
            pltpu.VMEM((G, 1, 2 * H), jnp.float32),
            pltpu.SemaphoreType.DMA((2,)),
        ],
    )(scat, t8, wl, wr, bl)


@jax.jit
def kernel(input_nodes, input_edges, node_emb, graph_token_emb, W_l, b_l, W_r):
    input_nodes = input_nodes.astype(jnp.int32)
    edges = input_edges.astype(jnp.int32)

    scat = _sc_hist()(input_nodes, edges).reshape(G, ROWS, NPAD)

    t8 = jnp.concatenate(
        [node_emb, graph_token_emb, jnp.zeros((3, H), jnp.float32)], axis=0)
    gnf, gef = _tc_dense(scat, t8, W_l, W_r, b_l.reshape(1, H))
    return jnp.swapaxes(gnf, 0, 1), jnp.swapaxes(gef, 0, 1)
